# baseline (speedup 1.0000x reference)
import functools

import jax
import jax.numpy as jnp
from jax import lax
from jax.experimental import pallas as pl
from jax.experimental.pallas import tpu as pltpu
from jax.experimental.pallas import tpu_sc as plsc

_D = 128
_L = 16
_NC = 2
_NS = 16
_NW = _NC * _NS
_C = 200


def _sc_permute(x_rows, idx):
    n = x_rows.shape[0]
    rows_w = n // _NW
    nchunk = rows_w // _C

    mesh = plsc.VectorSubcoreMesh(core_axis_name="c", subcore_axis_name="s")

    @functools.partial(
        pl.kernel,
        mesh=mesh,
        out_type=jax.ShapeDtypeStruct((n, _D), jnp.float32),
        compiler_params=pltpu.CompilerParams(needs_layout_passes=False),
        scratch_types=[
            pltpu.VMEM((_D,), jnp.int32),
            pltpu.VMEM((_C, _D), jnp.float32),
            pltpu.VMEM((_C, _D), jnp.float32),
        ],
    )
    def k(x_hbm, idx_hbm, out_hbm, idx_v, in_v, out_v):
        wid = lax.axis_index("s") * _NC + lax.axis_index("c")
        base = wid * rows_w

        pltpu.sync_copy(idx_hbm, idx_v)
        idx_vecs = [idx_v[pl.ds(_L * j, _L)] for j in range(_D // _L)]

        def chunk_body(g, _):
            start = base + g * _C
            pltpu.sync_copy(x_hbm.at[pl.ds(start, _C)], in_v)

            def row_body(r, _):
                row_v = jnp.full((_L,), r, jnp.int32)
                for j in range(_D // _L):
                    v = plsc.load_gather(in_v, [row_v, idx_vecs[j]])
                    out_v[r, pl.ds(_L * j, _L)] = v
                return 0

            lax.fori_loop(0, _C, row_body, 0)
            pltpu.sync_copy(out_v, out_hbm.at[pl.ds(start, _C)])
            return 0

        lax.fori_loop(0, nchunk, chunk_body, 0)

    return k(x_rows, idx)


def kernel(x, indexes):
    b, s, d = x.shape
    x_rows = x.reshape(b * s, d)
    idx = indexes.astype(jnp.int32)
    out = _sc_permute(x_rows, idx)
    return out.reshape(b, s, d)


# speedup vs baseline: 1.1551x; 1.1551x over previous
import functools

import jax
import jax.numpy as jnp
from jax import lax
from jax.experimental import pallas as pl
from jax.experimental.pallas import tpu as pltpu
from jax.experimental.pallas import tpu_sc as plsc

_D = 128
_L = 16
_NC = 2
_NS = 16
_NW = _NC * _NS
_C = 200


def _sc_permute(x_rows, idx):
    n = x_rows.shape[0]
    rows_w = n // _NW
    nchunk = rows_w // _C

    mesh = plsc.VectorSubcoreMesh(core_axis_name="c", subcore_axis_name="s")

    @functools.partial(
        pl.kernel,
        mesh=mesh,
        out_type=jax.ShapeDtypeStruct((n, _D), jnp.float32),
        compiler_params=pltpu.CompilerParams(needs_layout_passes=False),
        scratch_types=[
            pltpu.VMEM((_D,), jnp.int32),
            pltpu.VMEM((2, _C, _D), jnp.float32),
            pltpu.VMEM((2, _C, _D), jnp.float32),
            pltpu.SemaphoreType.DMA,
            pltpu.SemaphoreType.DMA,
            pltpu.SemaphoreType.DMA,
            pltpu.SemaphoreType.DMA,
        ],
    )
    def k(x_hbm, idx_hbm, out_hbm, idx_v, in_v, out_v, si0, si1, so0, so1):
        wid = lax.axis_index("s") * _NC + lax.axis_index("c")
        base = wid * rows_w

        pltpu.sync_copy(idx_hbm, idx_v)
        idx_vecs = [idx_v[pl.ds(_L * j, _L)] for j in range(_D // _L)]
        sin = (si0, si1)
        sout = (so0, so1)

        def rows(g):
            return pl.ds(base + g * _C, _C)

        in_cp = [
            pltpu.async_copy(x_hbm.at[rows(0)], in_v.at[0], si0),
            pltpu.async_copy(x_hbm.at[rows(1)], in_v.at[1], si1),
        ]
        out_cp = [None, None]
        for g in range(nchunk):
            b = g & 1
            in_cp[b].wait()
            if out_cp[b] is not None:
                out_cp[b].wait()

            def row_body(r, _):
                row_v = jnp.full((_L,), r, jnp.int32)
                for j in range(_D // _L):
                    v = plsc.load_gather(in_v.at[b], [row_v, idx_vecs[j]])
                    out_v[b, r, pl.ds(_L * j, _L)] = v
                return 0

            lax.fori_loop(0, _C, row_body, 0)
            out_cp[b] = pltpu.async_copy(out_v.at[b], out_hbm.at[rows(g)], sout[b])
            if g + 2 < nchunk:
                in_cp[b] = pltpu.async_copy(x_hbm.at[rows(g + 2)], in_v.at[b], sin[b])
        out_cp[0].wait()
        out_cp[1].wait()

    return k(x_rows, idx)


def kernel(x, indexes):
    b, s, d = x.shape
    x_rows = x.reshape(b * s, d)
    idx = indexes.astype(jnp.int32)
    out = _sc_permute(x_rows, idx)
    return out.reshape(b, s, d)


# speedup vs baseline: 1.5050x; 1.3029x over previous
import functools

import jax
import jax.numpy as jnp
from jax import lax
from jax.experimental import pallas as pl
from jax.experimental.pallas import tpu as pltpu
from jax.experimental.pallas import tpu_sc as plsc

_D = 128
_L = 16
_NC = 2
_NS = 16
_NW = _NC * _NS
_C = 200


def _sc_permute(x_rows, idx):
    n = x_rows.shape[0]
    rows_w = n // _NW
    nchunk = rows_w // _C

    mesh = plsc.VectorSubcoreMesh(core_axis_name="c", subcore_axis_name="s")

    @functools.partial(
        pl.kernel,
        mesh=mesh,
        out_type=jax.ShapeDtypeStruct((n, _D), jnp.float32),
        compiler_params=pltpu.CompilerParams(needs_layout_passes=False),
        scratch_types=[
            pltpu.VMEM((_D,), jnp.int32),
            pltpu.VMEM((2, _C, _D), jnp.float32),
            pltpu.VMEM((2, _C, _D), jnp.float32),
            pltpu.SemaphoreType.DMA,
            pltpu.SemaphoreType.DMA,
            pltpu.SemaphoreType.DMA,
            pltpu.SemaphoreType.DMA,
        ],
    )
    def k(x_hbm, idx_hbm, out_hbm, idx_v, in_v, out_v, si0, si1, so0, so1):
        wid = lax.axis_index("s") * _NC + lax.axis_index("c")
        base = wid * rows_w

        pltpu.sync_copy(idx_hbm, idx_v)
        idx_vecs = [idx_v[pl.ds(_L * j, _L)] for j in range(_D // _L)]
        sin = (si0, si1)
        sout = (so0, so1)

        def rows(g):
            return pl.ds(base + g * _C, _C)

        in_cp = [
            pltpu.async_copy(x_hbm.at[rows(0)], in_v.at[0], si0),
            pltpu.async_copy(x_hbm.at[rows(1)], in_v.at[1], si1),
        ]
        out_cp = [None, None]
        for g in range(nchunk):
            b = g & 1
            in_cp[b].wait()
            if out_cp[b] is not None:
                out_cp[b].wait()

            @plsc.parallel_loop(0, _C, unroll=4)
            def row_body(r):
                row_v = jnp.full((_L,), r, jnp.int32)
                for j in range(_D // _L):
                    v = plsc.load_gather(in_v.at[b], [row_v, idx_vecs[j]])
                    out_v[b, r, pl.ds(_L * j, _L)] = v
            out_cp[b] = pltpu.async_copy(out_v.at[b], out_hbm.at[rows(g)], sout[b])
            if g + 2 < nchunk:
                in_cp[b] = pltpu.async_copy(x_hbm.at[rows(g + 2)], in_v.at[b], sin[b])
        out_cp[0].wait()
        out_cp[1].wait()

    return k(x_rows, idx)


def kernel(x, indexes):
    b, s, d = x.shape
    x_rows = x.reshape(b * s, d)
    idx = indexes.astype(jnp.int32)
    out = _sc_permute(x_rows, idx)
    return out.reshape(b, s, d)


# speedup vs baseline: 1.5225x; 1.0116x over previous
import functools

import jax
import jax.numpy as jnp
from jax import lax
from jax.experimental import pallas as pl
from jax.experimental.pallas import tpu as pltpu
from jax.experimental.pallas import tpu_sc as plsc

_D = 128
_L = 16
_NC = 2
_NS = 16
_NW = _NC * _NS
_C = 80
_NBUF = 5


def _sc_permute(x_rows, idx):
    n = x_rows.shape[0]
    rows_w = n // _NW
    nchunk = rows_w // _C
    ngroups = nchunk // _NBUF

    mesh = plsc.VectorSubcoreMesh(core_axis_name="c", subcore_axis_name="s")

    @functools.partial(
        pl.kernel,
        mesh=mesh,
        out_type=jax.ShapeDtypeStruct((n, _D), jnp.float32),
        compiler_params=pltpu.CompilerParams(needs_layout_passes=False),
        scratch_types=[pltpu.VMEM((_D,), jnp.int32)]
        + [pltpu.VMEM((_C, _D), jnp.float32)] * (2 * _NBUF)
        + [pltpu.SemaphoreType.DMA] * (2 * _NBUF),
    )
    def k(x_hbm, idx_hbm, out_hbm, idx_v, *bufs_and_sems):
        in_bufs = bufs_and_sems[0:_NBUF]
        out_bufs = bufs_and_sems[_NBUF:2 * _NBUF]
        sin = bufs_and_sems[2 * _NBUF:3 * _NBUF]
        sout = bufs_and_sems[3 * _NBUF:4 * _NBUF]

        wid = lax.axis_index("s") * _NC + lax.axis_index("c")
        base = wid * rows_w

        pltpu.sync_copy(idx_hbm, idx_v)
        idx_vecs = [idx_v[pl.ds(_L * j, _L)] for j in range(_D // _L)]

        def rows(g):
            return pl.ds(base + g * _C, _C)

        for b in range(_NBUF):
            pltpu.async_copy(x_hbm.at[rows(b)], in_bufs[b], sin[b])

        def group_body(gi, _):
            g0 = gi * _NBUF
            for b in range(_NBUF):
                g = g0 + b
                in_b = in_bufs[b]
                out_b = out_bufs[b]

                pltpu.make_async_copy(x_hbm.at[rows(g)], in_b, sin[b]).wait()

                @pl.when(gi > 0)
                def _wait_out():
                    pltpu.make_async_copy(
                        out_b, out_hbm.at[rows(g)], sout[b]
                    ).wait()

                @plsc.parallel_loop(0, _C, unroll=4)
                def row_body(r):
                    row_v = jnp.full((_L,), r, jnp.int32)
                    for j in range(_D // _L):
                        v = plsc.load_gather(in_b, [row_v, idx_vecs[j]])
                        out_b[r, pl.ds(_L * j, _L)] = v

                pltpu.async_copy(out_b, out_hbm.at[rows(g)], sout[b])

                @pl.when(g + _NBUF < nchunk)
                def _issue_in():
                    pltpu.async_copy(x_hbm.at[rows(g + _NBUF)], in_b, sin[b])
            return 0

        lax.fori_loop(0, ngroups, group_body, 0)

        for b in range(_NBUF):
            g_last = (ngroups - 1) * _NBUF + b
            pltpu.make_async_copy(
                out_bufs[b], out_hbm.at[rows(g_last)], sout[b]
            ).wait()

    return k(x_rows, idx)


def kernel(x, indexes):
    b, s, d = x.shape
    x_rows = x.reshape(b * s, d)
    idx = indexes.astype(jnp.int32)
    out = _sc_permute(x_rows, idx)
    return out.reshape(b, s, d)
